# baseline (speedup 1.0000x reference)
import jax
import jax.numpy as jnp
from jax import lax
from jax.experimental import pallas as pl
from jax.experimental.pallas import tpu as pltpu
from jax.experimental.pallas import tpu_sc as plsc

NUM_FIELDS = 26
FIELD_SIZE = 100000
BATCH = 16384
NUM_WORKERS = 32
B_PER_W = BATCH // NUM_WORKERS
CHUNK = 128
N_CHUNKS = B_PER_W // CHUNK
LANES = 16
PREFIX = 2599936


def _sc_kernel(xt_hbm, ta_hbm, tb_hbm, bias_hbm, out_hbm,
               idx_v, val_v, acc_v, bias_v, sem):
    wid = lax.axis_index("s") * 2 + lax.axis_index("c")
    base = wid * B_PER_W

    pltpu.sync_copy(bias_hbm, bias_v)

    pltpu.sync_copy(xt_hbm.at[:, pl.ds(base, B_PER_W)], idx_v)

    @pl.loop(0, NUM_FIELDS - 1)
    def _fire(f):
        tview = ta_hbm.at[pl.ds(f * FIELD_SIZE, FIELD_SIZE)]

        @pl.loop(0, N_CHUNKS)
        def _fire_chunk(q):
            pltpu.async_copy(
                tview.at[idx_v.at[f, pl.ds(q * CHUNK, CHUNK)]],
                val_v.at[f, pl.ds(q * CHUNK, CHUNK)],
                sem,
            )

    @pl.loop(0, N_CHUNKS)
    def _fire_last(q):
        pltpu.async_copy(
            tb_hbm.at[idx_v.at[NUM_FIELDS - 1, pl.ds(q * CHUNK, CHUNK)]],
            val_v.at[NUM_FIELDS - 1, pl.ds(q * CHUNK, CHUNK)],
            sem,
        )

    @pl.loop(0, NUM_FIELDS * N_CHUNKS)
    def _drain(i):
        pltpu.make_async_copy(
            ta_hbm.at[pl.ds(0, CHUNK)],
            val_v.at[0, pl.ds(0, CHUNK)],
            sem,
        ).wait()

    b = bias_v[...]

    @pl.loop(0, B_PER_W, step=LANES)
    def _reduce(j):
        acc = jnp.full((LANES,), 0.0, jnp.float32)
        for f in range(NUM_FIELDS):
            acc = acc + val_v[f, pl.ds(j, LANES)]
        acc_v[pl.ds(j, LANES)] = 1.0 / (1.0 + jnp.exp(-(acc + b)))

    pltpu.sync_copy(acc_v, out_hbm.at[pl.ds(base, B_PER_W)])


@jax.jit
def kernel(x, table, bias):
    xt = x.astype(jnp.int32).T
    ta = table[:PREFIX, 0]
    tb = table[(NUM_FIELDS - 1) * FIELD_SIZE:, 0]
    bias_lanes = jnp.broadcast_to(bias, (LANES,))

    mesh = plsc.VectorSubcoreMesh(core_axis_name="c", subcore_axis_name="s")
    k = pl.kernel(
        _sc_kernel,
        out_type=jax.ShapeDtypeStruct((BATCH,), jnp.float32),
        mesh=mesh,
        scratch_types=[
            pltpu.VMEM((NUM_FIELDS, B_PER_W), jnp.int32),
            pltpu.VMEM((NUM_FIELDS, B_PER_W), jnp.float32),
            pltpu.VMEM((B_PER_W,), jnp.float32),
            pltpu.VMEM((LANES,), jnp.float32),
            pltpu.SemaphoreType.DMA,
        ],
    )
    return k(xt, ta, tb, bias_lanes)


# speedup vs baseline: 1.0010x; 1.0010x over previous
import jax
import jax.numpy as jnp
from jax import lax
from jax.experimental import pallas as pl
from jax.experimental.pallas import tpu as pltpu
from jax.experimental.pallas import tpu_sc as plsc

NUM_FIELDS = 26
FIELD_SIZE = 100000
BATCH = 16384
NUM_WORKERS = 32
B_PER_W = BATCH // NUM_WORKERS
CHUNK = 128
N_CHUNKS = B_PER_W // CHUNK
LANES = 16
PREFIX = 2599936
NVAL = NUM_FIELDS * B_PER_W


def _sc_kernel(xt_hbm, ta_hbm, tb_hbm, bias_hbm, out_hbm,
               idx_v, val_v, acc_v, bias_v, sem):
    wid = lax.axis_index("s") * 2 + lax.axis_index("c")
    base = wid * B_PER_W

    pltpu.sync_copy(bias_hbm, bias_v)

    pltpu.sync_copy(xt_hbm.at[:, pl.ds(base, B_PER_W)], idx_v)

    @pl.loop(0, NUM_FIELDS - 1)
    def _fire(f):
        tview = ta_hbm.at[pl.ds(f * FIELD_SIZE, FIELD_SIZE)]
        for q in range(N_CHUNKS):
            pltpu.async_copy(
                tview.at[idx_v.at[f, pl.ds(q * CHUNK, CHUNK)]],
                val_v.at[pl.ds(f * B_PER_W + q * CHUNK, CHUNK)],
                sem,
            )

    for q in range(N_CHUNKS):
        pltpu.async_copy(
            tb_hbm.at[idx_v.at[NUM_FIELDS - 1, pl.ds(q * CHUNK, CHUNK)]],
            val_v.at[pl.ds((NUM_FIELDS - 1) * B_PER_W + q * CHUNK, CHUNK)],
            sem,
        )

    pltpu.make_async_copy(ta_hbm.at[pl.ds(0, NVAL)], val_v, sem).wait()

    b = bias_v[...]

    @pl.loop(0, B_PER_W, step=LANES)
    def _reduce(j):
        acc = jnp.full((LANES,), 0.0, jnp.float32)
        for f in range(NUM_FIELDS):
            acc = acc + val_v[pl.ds(f * B_PER_W + j, LANES)]
        acc_v[pl.ds(j, LANES)] = 1.0 / (1.0 + jnp.exp(-(acc + b)))

    pltpu.sync_copy(acc_v, out_hbm.at[pl.ds(base, B_PER_W)])


@jax.jit
def kernel(x, table, bias):
    xt = x.astype(jnp.int32).T
    ta = table[:PREFIX, 0]
    tb = table[(NUM_FIELDS - 1) * FIELD_SIZE:, 0]
    bias_lanes = jnp.broadcast_to(bias, (LANES,))

    mesh = plsc.VectorSubcoreMesh(core_axis_name="c", subcore_axis_name="s")
    k = pl.kernel(
        _sc_kernel,
        out_type=jax.ShapeDtypeStruct((BATCH,), jnp.float32),
        mesh=mesh,
        scratch_types=[
            pltpu.VMEM((NUM_FIELDS, B_PER_W), jnp.int32),
            pltpu.VMEM((NVAL,), jnp.float32),
            pltpu.VMEM((B_PER_W,), jnp.float32),
            pltpu.VMEM((LANES,), jnp.float32),
            pltpu.SemaphoreType.DMA,
        ],
    )
    return k(xt, ta, tb, bias_lanes)


# speedup vs baseline: 1.0269x; 1.0258x over previous
import jax
import jax.numpy as jnp
from jax import lax
from jax.experimental import pallas as pl
from jax.experimental.pallas import tpu as pltpu
from jax.experimental.pallas import tpu_sc as plsc

NUM_FIELDS = 26
FIELD_SIZE = 100000
BATCH = 16384
NUM_WORKERS = 32
B_PER_W = BATCH // NUM_WORKERS
CHUNK = 128
N_CHUNKS = B_PER_W // CHUNK
LANES = 16
PREFIX = 2599936
NVAL = NUM_FIELDS * B_PER_W


def _sc_kernel(xt_hbm, ta_hbm, tb_hbm, bias_hbm, out_hbm,
               idx_v, val_v, acc_v, bias_v, sem):
    wid = lax.axis_index("s") * 2 + lax.axis_index("c")
    base = wid * B_PER_W

    pltpu.sync_copy(bias_hbm, bias_v)

    pltpu.sync_copy(xt_hbm.at[:, pl.ds(base, B_PER_W)], idx_v)

    @pl.loop(0, NUM_FIELDS - 1)
    def _fire(f):
        tview = ta_hbm.at[pl.ds(f * FIELD_SIZE, FIELD_SIZE)]
        for q in range(N_CHUNKS):
            pltpu.async_copy(
                tview.at[idx_v.at[f, pl.ds(q * CHUNK, CHUNK)]],
                val_v.at[pl.ds(f * B_PER_W + q * CHUNK, CHUNK)],
                sem,
            )

    for q in range(N_CHUNKS):
        pltpu.async_copy(
            tb_hbm.at[idx_v.at[NUM_FIELDS - 1, pl.ds(q * CHUNK, CHUNK)]],
            val_v.at[pl.ds((NUM_FIELDS - 1) * B_PER_W + q * CHUNK, CHUNK)],
            sem,
        )

    pltpu.make_async_copy(ta_hbm.at[pl.ds(0, NVAL)], val_v, sem).wait()

    b = plsc.load_gather(bias_v, [jax.lax.iota(jnp.int32, 16) * 0])

    @pl.loop(0, B_PER_W, step=LANES)
    def _reduce(j):
        acc = jnp.full((LANES,), 0.0, jnp.float32)
        for f in range(NUM_FIELDS):
            acc = acc + val_v[pl.ds(f * B_PER_W + j, LANES)]
        acc_v[pl.ds(j, LANES)] = 1.0 / (1.0 + jnp.exp(-(acc + b)))

    pltpu.sync_copy(acc_v, out_hbm.at[pl.ds(base, B_PER_W)])


@jax.jit
def kernel(x, table, bias):
    xt = x.astype(jnp.int32).T
    ta = table[:PREFIX, 0]
    tb = table[(NUM_FIELDS - 1) * FIELD_SIZE:, 0]

    mesh = plsc.VectorSubcoreMesh(core_axis_name="c", subcore_axis_name="s")
    k = pl.kernel(
        _sc_kernel,
        out_type=jax.ShapeDtypeStruct((BATCH,), jnp.float32),
        mesh=mesh,
        compiler_params=pltpu.CompilerParams(needs_layout_passes=False),
        scratch_types=[
            pltpu.VMEM((NUM_FIELDS, B_PER_W), jnp.int32),
            pltpu.VMEM((NVAL,), jnp.float32),
            pltpu.VMEM((B_PER_W,), jnp.float32),
            pltpu.VMEM((1,), jnp.float32),
            pltpu.SemaphoreType.DMA,
        ],
    )
    return k(xt, ta, tb, bias)
